# baseline (speedup 1.0000x reference)
import jax
import jax.numpy as jnp
from jax.experimental import pallas as pl
from jax.experimental.pallas import tpu as pltpu


def _conv(x, w, b):
    out = jax.lax.conv_general_dilated(
        x, w, (1, 1), 'VALID', dimension_numbers=('NCHW', 'OIHW', 'NCHW'))
    return out + b[None, :, None, None]


def _maxpool(x, k):
    return jax.lax.reduce_window(
        x, -jnp.inf, jax.lax.max, (1, 1, k, k), (1, 1, k, k), 'VALID')


def _deconv(x, w, b, stride):
    w2 = jnp.flip(jnp.transpose(w, (1, 0, 2, 3)), axis=(2, 3))
    k = w.shape[2]
    out = jax.lax.conv_general_dilated(
        x, w2, window_strides=(1, 1), padding=[(k - 1, k - 1), (k - 1, k - 1)],
        lhs_dilation=(stride, stride), dimension_numbers=('NCHW', 'OIHW', 'NCHW'))
    return out + b[None, :, None, None]


_D1_TAPS = {0: [(1, -1), (6, 0)], 1: [(0, -1), (5, 0)],
            2: [(4, 0)], 3: [(3, 0)], 4: [(2, 0)]}
_D2_TAPS = {0: [(1, -1), (4, 0)], 1: [(0, -1), (3, 0)], 2: [(2, 0)]}
_D3_TAPS = {0: [(0, -1), (2, 0)], 1: [(1, 0)]}


def _d1_body(inp_ref, w2_ref, b_ref, out_ref):
    for co in range(16):
        bias = b_ref[0, co]
        for ry in range(5):
            for rx in range(5):
                acc = jnp.zeros((8, 8, 64), jnp.float32)
                for (uy, jy) in _D1_TAPS[ry]:
                    for (ux, jx) in _D1_TAPS[rx]:
                        sy, sx = jy + 1, jx + 1

                        def body(ci, a, co=co, uy=uy, ux=ux, sy=sy, sx=sx):
                            return a + w2_ref[co, ci, uy, ux] * \
                                inp_ref[ci, sy:sy + 8, sx:sx + 8, :]
                        acc = jax.lax.fori_loop(0, 32, body, acc)
                out_ref[co, ry, rx, :, :, :] = jnp.maximum(acc + bias, 0.0)


def _d2_body(inp_ref, w2_ref, b_ref, out_ref):
    bias = b_ref[pl.program_id(0), 0]
    for ry in range(3):
        for rx in range(3):
            acc = jnp.zeros((38, 38, 64), jnp.float32)
            for (uy, jy) in _D2_TAPS[ry]:
                for (ux, jx) in _D2_TAPS[rx]:
                    sy, sx = jy + 1, jx + 1

                    def body(ci, a, uy=uy, ux=ux, sy=sy, sx=sx):
                        return a + w2_ref[0, ci, uy, ux] * \
                            inp_ref[ci, sy:sy + 38, sx:sx + 38, :]
                    acc = jax.lax.fori_loop(0, 16, body, acc)
            out_ref[0, ry, rx, :, :, :] = jnp.maximum(acc + bias, 0.0)


def _d3_body(inp_ref, w2_ref, b_ref, out_ref):
    for co in range(4):
        bias = b_ref[0, co]
        for ry in range(2):
            for rx in range(2):
                acc = jnp.zeros((114, 114), jnp.float32)
                for (uy, jy) in _D3_TAPS[ry]:
                    for (ux, jx) in _D3_TAPS[rx]:
                        sy, sx = jy + 1, jx + 1
                        for ci in range(8):
                            acc = acc + w2_ref[co, ci, uy, ux] * \
                                inp_ref[0, ci, sy:sy + 114, sx:sx + 114]
                out_ref[0, co, ry, rx, :, :] = jax.nn.sigmoid(acc + bias)


def _w2(w):
    return jnp.flip(jnp.transpose(w, (1, 0, 2, 3)), axis=(2, 3))


def _decoder(d, p):
    inp = jnp.transpose(d.reshape(64, 32, 7, 7), (1, 2, 3, 0))
    inp = jnp.pad(inp, ((0, 0), (1, 1), (1, 1), (0, 0)))
    p1 = pl.pallas_call(
        _d1_body,
        out_shape=jax.ShapeDtypeStruct((16, 5, 5, 8, 8, 64), jnp.float32),
    )(inp, _w2(p['dec_d1_w']), p['dec_d1_b'][None, :])
    h1 = jnp.transpose(p1, (0, 3, 1, 4, 2, 5)).reshape(16, 40, 40, 64)
    h1 = jnp.pad(h1[:, :37, :37, :], ((0, 0), (1, 1), (1, 1), (0, 0)))
    p2 = pl.pallas_call(
        _d2_body,
        grid=(8,),
        in_specs=[
            pl.BlockSpec((16, 39, 39, 64), lambda c: (0, 0, 0, 0)),
            pl.BlockSpec((1, 16, 5, 5), lambda c: (c, 0, 0, 0)),
            pl.BlockSpec((8, 1), lambda c: (0, 0)),
        ],
        out_specs=pl.BlockSpec((1, 3, 3, 38, 38, 64),
                               lambda c: (c, 0, 0, 0, 0, 0)),
        out_shape=jax.ShapeDtypeStruct((8, 3, 3, 38, 38, 64), jnp.float32),
    )(h1, _w2(p['dec_d2_w']), p['dec_d2_b'][:, None])
    h2 = jnp.transpose(p2, (5, 0, 3, 1, 4, 2)).reshape(64, 8, 114, 114)
    h2 = jnp.pad(h2[:, :, :113, :113], ((0, 0), (0, 0), (1, 1), (1, 1)))
    p3 = pl.pallas_call(
        _d3_body,
        grid=(64,),
        in_specs=[
            pl.BlockSpec((1, 8, 115, 115), lambda b: (b, 0, 0, 0)),
            pl.BlockSpec((4, 8, 3, 3), lambda b: (0, 0, 0, 0)),
            pl.BlockSpec((1, 4), lambda b: (0, 0)),
        ],
        out_specs=pl.BlockSpec((1, 4, 2, 2, 114, 114),
                               lambda b: (b, 0, 0, 0, 0, 0)),
        out_shape=jax.ShapeDtypeStruct((64, 4, 2, 2, 114, 114), jnp.float32),
    )(h2, _w2(p['dec_d3_w']), p['dec_d3_b'][None, :])
    out = jnp.transpose(p3, (0, 1, 4, 2, 5, 3)).reshape(64, 4, 228, 228)
    return out[:, :, :227, :227]


def _dotT(a, w):
    return jax.lax.dot_general(
        a, w, (((1,), (1,)), ((), ())), preferred_element_type=jnp.float32)


def _mid_body(h_ref, l1w_ref, l1b_ref, l2w_ref, l2b_ref, e_ref,
              d1w_ref, d1b_ref, d2w_ref, d2b_ref, inds_ref, dec_ref):
    h = h_ref[:]
    a = jnp.maximum(_dotT(h, l1w_ref[:]) + l1b_ref[:], 0.0)
    xe = _dotT(a, l2w_ref[:]) + l2b_ref[:]
    E = e_ref[:]
    x_norm = jnp.sum(xe * xe, axis=1, keepdims=True)
    e_norm = jnp.sum(E * E, axis=1, keepdims=True)
    prod = _dotT(xe, E)
    dis = (x_norm + e_norm.T) - 2.0 * prod
    m = jnp.min(dis, axis=1, keepdims=True)
    ii = jax.lax.broadcasted_iota(jnp.int32, dis.shape, 1)
    inds = jnp.min(jnp.where(dis == m, ii, jnp.int32(2 ** 30)), axis=1)
    inds_ref[0, :] = inds
    onehot = (ii == inds[:, None]).astype(jnp.float32)
    xq = jax.lax.dot_general(
        onehot, E, (((1,), (0,)), ((), ())),
        preferred_element_type=jnp.float32)
    d1 = jnp.maximum(_dotT(xq, d1w_ref[:]) + d1b_ref[:], 0.0)
    dec_ref[:] = jnp.maximum(_dotT(d1, d2w_ref[:]) + d2b_ref[:], 0.0)


def _mid(h, p):
    B = h.shape[0]
    inds2d, dec = pl.pallas_call(
        _mid_body,
        out_shape=(
            jax.ShapeDtypeStruct((1, B), jnp.int32),
            jax.ShapeDtypeStruct((B, 1568), jnp.float32),
        ),
    )(h, p['enc_l1_w'], p['enc_l1_b'][None, :], p['enc_l2_w'],
      p['enc_l2_b'][None, :], p['codebook'], p['dec_l1_w'],
      p['dec_l1_b'][None, :], p['dec_l2_w'], p['dec_l2_b'][None, :])
    return inds2d[0], dec


def kernel(x, params):
    p = params
    h = jax.nn.relu(_maxpool(_conv(x, p['enc_c1_w'], p['enc_c1_b']), 2))
    h = jax.nn.relu(_maxpool(_conv(h, p['enc_c2_w'], p['enc_c2_b']), 3))
    h = jax.nn.relu(_maxpool(_conv(h, p['enc_c3_w'], p['enc_c3_b']), 5))
    h = h.reshape(-1, 32 * 7 * 7)
    embed_inds, d = _mid(h, p)
    x_reconstructed = _decoder(d, p)
    return (x_reconstructed, embed_inds)


# speedup vs baseline: 2.7845x; 2.7845x over previous
import jax
import jax.numpy as jnp
from jax.experimental import pallas as pl
from jax.experimental.pallas import tpu as pltpu


def _conv(x, w, b):
    out = jax.lax.conv_general_dilated(
        x, w, (1, 1), 'VALID', dimension_numbers=('NCHW', 'OIHW', 'NCHW'))
    return out + b[None, :, None, None]


def _maxpool(x, k):
    return jax.lax.reduce_window(
        x, -jnp.inf, jax.lax.max, (1, 1, k, k), (1, 1, k, k), 'VALID')


_D1_TAPS = {0: [(1, -1), (6, 0)], 1: [(0, -1), (5, 0)],
            2: [(4, 0)], 3: [(3, 0)], 4: [(2, 0)]}
_D2_TAPS = {0: [(1, -1), (4, 0)], 1: [(0, -1), (3, 0)], 2: [(2, 0)]}
_D3_TAPS = {0: [(0, -1), (2, 0)], 1: [(1, 0)]}


def _d1_body(ia_ref, ib_ref, wv_ref, b_ref, out_ref):
    for ry in range(5):
        for rx in range(5):
            taps = [(uy, jy + 1, ux, jx + 1)
                    for (uy, jy) in _D1_TAPS[ry] for (ux, jx) in _D1_TAPS[rx]]

            def body(cp, accs, taps=taps):
                new = list(accs)
                for (uy, sy, ux, sx) in taps:
                    var = ia_ref if sx == 0 else ib_ref
                    v = var[cp, sy:sy + 8, :, :]
                    for co in range(16):
                        new[co] = new[co] + wv_ref[co, cp, uy, ux, :] * v
                return tuple(new)
            accs = jax.lax.fori_loop(
                0, 16, body,
                tuple(jnp.zeros((8, 8, 128), jnp.float32) for _ in range(16)))
            for co in range(16):
                h = accs[co][:, :, 0:64] + accs[co][:, :, 64:128]
                out_ref[co, ry, rx, :, :, :] = jnp.maximum(h + b_ref[co, 0], 0.0)


def _d2_body(ia_ref, ib_ref, wv_ref, b_ref, out_ref):
    bias = b_ref[pl.program_id(0), 0]
    for ry in range(3):
        for rx in range(3):
            taps = [(uy, jy + 1, ux, jx + 1)
                    for (uy, jy) in _D2_TAPS[ry] for (ux, jx) in _D2_TAPS[rx]]

            def body(cp, acc, taps=taps):
                for (uy, sy, ux, sx) in taps:
                    var = ia_ref if sx == 0 else ib_ref
                    acc = acc + wv_ref[0, cp, uy, ux, :] * \
                        var[cp, sy:sy + 38, :, :]
                return acc
            acc = jax.lax.fori_loop(
                0, 8, body, jnp.zeros((38, 38, 128), jnp.float32))
            h = acc[:, :, 0:64] + acc[:, :, 64:128]
            out_ref[0, ry, rx, :, :, :] = jnp.maximum(h + bias, 0.0)


def _d3_body(i00_ref, i01_ref, i10_ref, i11_ref, w2_ref, b_ref, out_ref):
    variants = {(0, 0): i00_ref, (0, 1): i01_ref,
                (1, 0): i10_ref, (1, 1): i11_ref}
    for img in range(4):
        for ry in range(2):
            for rx in range(2):
                taps = [(uy, jy + 1, ux, jx + 1)
                        for (uy, jy) in _D3_TAPS[ry]
                        for (ux, jx) in _D3_TAPS[rx]]

                def body(ci, accs, taps=taps, img=img):
                    new = list(accs)
                    for (uy, sy, ux, sx) in taps:
                        v = variants[(sy, sx)][img, ci, :, :]
                        for co in range(4):
                            new[co] = new[co] + w2_ref[co, ci, uy, ux] * v
                    return tuple(new)
                accs = jax.lax.fori_loop(
                    0, 8, body,
                    tuple(jnp.zeros((114, 114), jnp.float32) for _ in range(4)))
                for co in range(4):
                    out_ref[img, co, ry, rx, :, :] = jax.nn.sigmoid(
                        accs[co] + b_ref[co, 0])


def _w2(w):
    return jnp.flip(jnp.transpose(w, (1, 0, 2, 3)), axis=(2, 3))


def _pair_lanes(a, groups=2):
    g, c = groups, a.shape[0] // groups
    return jnp.transpose(
        a.reshape(g, c, *a.shape[1:]), (1, 2, 3, 0, 4)).reshape(
            c, a.shape[1], a.shape[2], g * a.shape[3])


def _pair_w(w2, groups=2):
    co, ci, kh, kw = w2.shape
    c = ci // groups
    wv = jnp.transpose(w2.reshape(co, groups, c, kh, kw), (0, 2, 3, 4, 1))
    return jnp.repeat(wv, 64, axis=4)


def _decoder(d, p):
    inp = jnp.transpose(d.reshape(64, 32, 7, 7), (1, 2, 3, 0))
    inp = jnp.pad(inp, ((0, 0), (1, 1), (1, 1), (0, 0)))
    inp = _pair_lanes(inp)
    w1 = _w2(p['dec_d1_w'])
    p1 = pl.pallas_call(
        _d1_body,
        out_shape=jax.ShapeDtypeStruct((16, 5, 5, 8, 8, 64), jnp.float32),
    )(inp[:, :, 0:8, :], inp[:, :, 1:9, :], _pair_w(w1),
      p['dec_d1_b'][:, None])
    h1 = jnp.transpose(p1, (0, 3, 1, 4, 2, 5)).reshape(16, 40, 40, 64)
    h1 = jnp.pad(h1[:, :37, :37, :], ((0, 0), (1, 1), (1, 1), (0, 0)))
    h1 = _pair_lanes(h1)
    w2d = _w2(p['dec_d2_w'])
    p2 = pl.pallas_call(
        _d2_body,
        grid=(8,),
        in_specs=[
            pl.BlockSpec((8, 39, 38, 128), lambda c: (0, 0, 0, 0)),
            pl.BlockSpec((8, 39, 38, 128), lambda c: (0, 0, 0, 0)),
            pl.BlockSpec((1, 8, 5, 5, 128), lambda c: (c, 0, 0, 0, 0)),
            pl.BlockSpec((8, 1), lambda c: (0, 0)),
        ],
        out_specs=pl.BlockSpec((1, 3, 3, 38, 38, 64),
                               lambda c: (c, 0, 0, 0, 0, 0)),
        out_shape=jax.ShapeDtypeStruct((8, 3, 3, 38, 38, 64), jnp.float32),
    )(h1[:, :, 0:38, :], h1[:, :, 1:39, :], _pair_w(w2d),
      p['dec_d2_b'][:, None])
    h2 = jnp.transpose(p2, (5, 0, 3, 1, 4, 2)).reshape(64, 8, 114, 114)
    h2 = jnp.pad(h2[:, :, :113, :113], ((0, 0), (0, 0), (1, 1), (1, 1)))
    ispec = pl.BlockSpec((4, 8, 114, 114), lambda b: (b, 0, 0, 0))
    p3 = pl.pallas_call(
        _d3_body,
        grid=(16,),
        in_specs=[
            ispec, ispec, ispec, ispec,
            pl.BlockSpec((4, 8, 3, 3), lambda b: (0, 0, 0, 0)),
            pl.BlockSpec((4, 1), lambda b: (0, 0)),
        ],
        out_specs=pl.BlockSpec((4, 4, 2, 2, 114, 114),
                               lambda b: (b, 0, 0, 0, 0, 0)),
        out_shape=jax.ShapeDtypeStruct((64, 4, 2, 2, 114, 114), jnp.float32),
    )(h2[:, :, 0:114, 0:114], h2[:, :, 0:114, 1:115],
      h2[:, :, 1:115, 0:114], h2[:, :, 1:115, 1:115],
      _w2(p['dec_d3_w']), p['dec_d3_b'][:, None])
    out = jnp.transpose(p3, (0, 1, 4, 2, 5, 3)).reshape(64, 4, 228, 228)
    return out[:, :, :227, :227]


def _dotT(a, w):
    return jax.lax.dot_general(
        a, w, (((1,), (1,)), ((), ())), preferred_element_type=jnp.float32)


def _mid_body(h_ref, l1w_ref, l1b_ref, l2w_ref, l2b_ref, e_ref,
              d1w_ref, d1b_ref, d2w_ref, d2b_ref, inds_ref, dec_ref):
    h = h_ref[:]
    a = jnp.maximum(_dotT(h, l1w_ref[:]) + l1b_ref[:], 0.0)
    xe = _dotT(a, l2w_ref[:]) + l2b_ref[:]
    E = e_ref[:]
    x_norm = jnp.sum(xe * xe, axis=1, keepdims=True)
    e_norm = jnp.sum(E * E, axis=1, keepdims=True)
    prod = _dotT(xe, E)
    dis = (x_norm + e_norm.T) - 2.0 * prod
    m = jnp.min(dis, axis=1, keepdims=True)
    ii = jax.lax.broadcasted_iota(jnp.int32, dis.shape, 1)
    inds = jnp.min(jnp.where(dis == m, ii, jnp.int32(2 ** 30)), axis=1)
    inds_ref[0, :] = inds
    onehot = (ii == inds[:, None]).astype(jnp.float32)
    xq = jax.lax.dot_general(
        onehot, E, (((1,), (0,)), ((), ())),
        preferred_element_type=jnp.float32)
    d1 = jnp.maximum(_dotT(xq, d1w_ref[:]) + d1b_ref[:], 0.0)
    dec_ref[:] = jnp.maximum(_dotT(d1, d2w_ref[:]) + d2b_ref[:], 0.0)


def _mid(h, p):
    B = h.shape[0]
    inds2d, dec = pl.pallas_call(
        _mid_body,
        out_shape=(
            jax.ShapeDtypeStruct((1, B), jnp.int32),
            jax.ShapeDtypeStruct((B, 1568), jnp.float32),
        ),
    )(h, p['enc_l1_w'], p['enc_l1_b'][None, :], p['enc_l2_w'],
      p['enc_l2_b'][None, :], p['codebook'], p['dec_l1_w'],
      p['dec_l1_b'][None, :], p['dec_l2_w'], p['dec_l2_b'][None, :])
    return inds2d[0], dec


def kernel(x, params):
    p = params
    h = jax.nn.relu(_maxpool(_conv(x, p['enc_c1_w'], p['enc_c1_b']), 2))
    h = jax.nn.relu(_maxpool(_conv(h, p['enc_c2_w'], p['enc_c2_b']), 3))
    h = jax.nn.relu(_maxpool(_conv(h, p['enc_c3_w'], p['enc_c3_b']), 5))
    h = h.reshape(-1, 32 * 7 * 7)
    embed_inds, d = _mid(h, p)
    x_reconstructed = _decoder(d, p)
    return (x_reconstructed, embed_inds)


# speedup vs baseline: 3.5845x; 1.2873x over previous
import jax
import jax.numpy as jnp
from jax.experimental import pallas as pl
from jax.experimental.pallas import tpu as pltpu


def _conv(x, w, b):
    out = jax.lax.conv_general_dilated(
        x, w, (1, 1), 'VALID', dimension_numbers=('NCHW', 'OIHW', 'NCHW'))
    return out + b[None, :, None, None]


def _maxpool(x, k):
    return jax.lax.reduce_window(
        x, -jnp.inf, jax.lax.max, (1, 1, k, k), (1, 1, k, k), 'VALID')


_D1_TAPS = {0: [(1, -1), (6, 0)], 1: [(0, -1), (5, 0)],
            2: [(4, 0)], 3: [(3, 0)], 4: [(2, 0)]}
_D2_TAPS = {0: [(1, -1), (4, 0)], 1: [(0, -1), (3, 0)], 2: [(2, 0)]}
_D3_TAPS = {0: [(0, -1), (2, 0)], 1: [(1, 0)]}


def _d1_body(ia_ref, ib_ref, wv_ref, b_ref, out_ref):
    for ry in range(5):
        for rx in range(5):
            taps = [(uy, jy + 1, ux, jx + 1)
                    for (uy, jy) in _D1_TAPS[ry] for (ux, jx) in _D1_TAPS[rx]]

            accs = [jnp.zeros((8, 8, 128), jnp.float32) for _ in range(16)]
            for cp in range(16):
                for (uy, sy, ux, sx) in taps:
                    var = ia_ref if sx == 0 else ib_ref
                    v = var[cp, sy:sy + 8, :, :]
                    for co in range(16):
                        accs[co] = accs[co] + wv_ref[co, cp, uy, ux, :] * v
            for co in range(16):
                h = accs[co][:, :, 0:64] + accs[co][:, :, 64:128]
                out_ref[co, ry, rx, :, :, :] = jnp.maximum(h + b_ref[co, 0], 0.0)


def _d2_body(ia_ref, ib_ref, wv_ref, b_ref, out_ref):
    bias = b_ref[pl.program_id(0), 0]
    for ry in range(3):
        for rx in range(3):
            taps = [(uy, jy + 1, ux, jx + 1)
                    for (uy, jy) in _D2_TAPS[ry] for (ux, jx) in _D2_TAPS[rx]]

            acc = jnp.zeros((38, 38, 128), jnp.float32)
            for cp in range(8):
                for (uy, sy, ux, sx) in taps:
                    var = ia_ref if sx == 0 else ib_ref
                    acc = acc + wv_ref[0, cp, uy, ux, :] * \
                        var[cp, sy:sy + 38, :, :]
            h = acc[:, :, 0:64] + acc[:, :, 64:128]
            out_ref[0, ry, rx, :, :, :] = jnp.maximum(h + bias, 0.0)


def _d3_body(i00_ref, i01_ref, i10_ref, i11_ref, w2_ref, b_ref, out_ref):
    variants = {(0, 0): i00_ref, (0, 1): i01_ref,
                (1, 0): i10_ref, (1, 1): i11_ref}
    for img in range(4):
        for ry in range(2):
            for rx in range(2):
                taps = [(uy, jy + 1, ux, jx + 1)
                        for (uy, jy) in _D3_TAPS[ry]
                        for (ux, jx) in _D3_TAPS[rx]]

                accs = [jnp.zeros((114, 114), jnp.float32) for _ in range(4)]
                for ci in range(8):
                    for (uy, sy, ux, sx) in taps:
                        v = variants[(sy, sx)][img, ci, :, :]
                        for co in range(4):
                            accs[co] = accs[co] + w2_ref[co, ci, uy, ux] * v
                for co in range(4):
                    out_ref[img, co, ry, rx, :, :] = jax.nn.sigmoid(
                        accs[co] + b_ref[co, 0])


def _w2(w):
    return jnp.flip(jnp.transpose(w, (1, 0, 2, 3)), axis=(2, 3))


def _pair_lanes(a, groups=2):
    g, c = groups, a.shape[0] // groups
    return jnp.transpose(
        a.reshape(g, c, *a.shape[1:]), (1, 2, 3, 0, 4)).reshape(
            c, a.shape[1], a.shape[2], g * a.shape[3])


def _pair_w(w2, groups=2):
    co, ci, kh, kw = w2.shape
    c = ci // groups
    wv = jnp.transpose(w2.reshape(co, groups, c, kh, kw), (0, 2, 3, 4, 1))
    return jnp.repeat(wv, 64, axis=4)


def _decoder(d, p):
    inp = jnp.transpose(d.reshape(64, 32, 7, 7), (1, 2, 3, 0))
    inp = jnp.pad(inp, ((0, 0), (1, 1), (1, 1), (0, 0)))
    inp = _pair_lanes(inp)
    w1 = _w2(p['dec_d1_w'])
    p1 = pl.pallas_call(
        _d1_body,
        out_shape=jax.ShapeDtypeStruct((16, 5, 5, 8, 8, 64), jnp.float32),
    )(inp[:, :, 0:8, :], inp[:, :, 1:9, :], _pair_w(w1),
      p['dec_d1_b'][:, None])
    h1 = jnp.transpose(p1, (0, 3, 1, 4, 2, 5)).reshape(16, 40, 40, 64)
    h1 = jnp.pad(h1[:, :37, :37, :], ((0, 0), (1, 1), (1, 1), (0, 0)))
    h1 = _pair_lanes(h1)
    w2d = _w2(p['dec_d2_w'])
    p2 = pl.pallas_call(
        _d2_body,
        grid=(8,),
        in_specs=[
            pl.BlockSpec((8, 39, 38, 128), lambda c: (0, 0, 0, 0)),
            pl.BlockSpec((8, 39, 38, 128), lambda c: (0, 0, 0, 0)),
            pl.BlockSpec((1, 8, 5, 5, 128), lambda c: (c, 0, 0, 0, 0)),
            pl.BlockSpec((8, 1), lambda c: (0, 0)),
        ],
        out_specs=pl.BlockSpec((1, 3, 3, 38, 38, 64),
                               lambda c: (c, 0, 0, 0, 0, 0)),
        out_shape=jax.ShapeDtypeStruct((8, 3, 3, 38, 38, 64), jnp.float32),
    )(h1[:, :, 0:38, :], h1[:, :, 1:39, :], _pair_w(w2d),
      p['dec_d2_b'][:, None])
    h2 = jnp.transpose(p2, (5, 0, 3, 1, 4, 2)).reshape(64, 8, 114, 114)
    h2 = jnp.pad(h2[:, :, :113, :113], ((0, 0), (0, 0), (1, 1), (1, 1)))
    ispec = pl.BlockSpec((4, 8, 114, 114), lambda b: (b, 0, 0, 0))
    p3 = pl.pallas_call(
        _d3_body,
        grid=(16,),
        in_specs=[
            ispec, ispec, ispec, ispec,
            pl.BlockSpec((4, 8, 3, 3), lambda b: (0, 0, 0, 0)),
            pl.BlockSpec((4, 1), lambda b: (0, 0)),
        ],
        out_specs=pl.BlockSpec((4, 4, 2, 2, 114, 114),
                               lambda b: (b, 0, 0, 0, 0, 0)),
        out_shape=jax.ShapeDtypeStruct((64, 4, 2, 2, 114, 114), jnp.float32),
    )(h2[:, :, 0:114, 0:114], h2[:, :, 0:114, 1:115],
      h2[:, :, 1:115, 0:114], h2[:, :, 1:115, 1:115],
      _w2(p['dec_d3_w']), p['dec_d3_b'][:, None])
    out = jnp.transpose(p3, (0, 1, 4, 2, 5, 3)).reshape(64, 4, 228, 228)
    return out[:, :, :227, :227]


def _dotT(a, w):
    return jax.lax.dot_general(
        a, w, (((1,), (1,)), ((), ())), preferred_element_type=jnp.float32)


def _mid_body(h_ref, l1w_ref, l1b_ref, l2w_ref, l2b_ref, e_ref,
              d1w_ref, d1b_ref, d2w_ref, d2b_ref, inds_ref, dec_ref):
    h = h_ref[:]
    a = jnp.maximum(_dotT(h, l1w_ref[:]) + l1b_ref[:], 0.0)
    xe = _dotT(a, l2w_ref[:]) + l2b_ref[:]
    E = e_ref[:]
    x_norm = jnp.sum(xe * xe, axis=1, keepdims=True)
    e_norm = jnp.sum(E * E, axis=1, keepdims=True)
    prod = _dotT(xe, E)
    dis = (x_norm + e_norm.T) - 2.0 * prod
    m = jnp.min(dis, axis=1, keepdims=True)
    ii = jax.lax.broadcasted_iota(jnp.int32, dis.shape, 1)
    inds = jnp.min(jnp.where(dis == m, ii, jnp.int32(2 ** 30)), axis=1)
    inds_ref[0, :] = inds
    onehot = (ii == inds[:, None]).astype(jnp.float32)
    xq = jax.lax.dot_general(
        onehot, E, (((1,), (0,)), ((), ())),
        preferred_element_type=jnp.float32)
    d1 = jnp.maximum(_dotT(xq, d1w_ref[:]) + d1b_ref[:], 0.0)
    dec_ref[:] = jnp.maximum(_dotT(d1, d2w_ref[:]) + d2b_ref[:], 0.0)


def _mid(h, p):
    B = h.shape[0]
    inds2d, dec = pl.pallas_call(
        _mid_body,
        out_shape=(
            jax.ShapeDtypeStruct((1, B), jnp.int32),
            jax.ShapeDtypeStruct((B, 1568), jnp.float32),
        ),
    )(h, p['enc_l1_w'], p['enc_l1_b'][None, :], p['enc_l2_w'],
      p['enc_l2_b'][None, :], p['codebook'], p['dec_l1_w'],
      p['dec_l1_b'][None, :], p['dec_l2_w'], p['dec_l2_b'][None, :])
    return inds2d[0], dec


def kernel(x, params):
    p = params
    h = jax.nn.relu(_maxpool(_conv(x, p['enc_c1_w'], p['enc_c1_b']), 2))
    h = jax.nn.relu(_maxpool(_conv(h, p['enc_c2_w'], p['enc_c2_b']), 3))
    h = jax.nn.relu(_maxpool(_conv(h, p['enc_c3_w'], p['enc_c3_b']), 5))
    h = h.reshape(-1, 32 * 7 * 7)
    embed_inds, d = _mid(h, p)
    x_reconstructed = _decoder(d, p)
    return (x_reconstructed, embed_inds)
